# baseline (speedup 1.0000x reference)
import jax
import jax.numpy as jnp
from jax import lax
from jax.experimental import pallas as pl
from jax.experimental.pallas import tpu as pltpu
from jax.experimental.pallas import tpu_sc as plsc

_N = 100000
_NW = 32
_R = 80
_NBLK = _N // _R
_NSTEP = -(-_NBLK // _NW)
_L = 16

_UNIT_MAP = (
    [(0, u) for u in range(4)] + [(1, u) for u in range(4)]
    + [(0, 4 + u) for u in range(6)] + [(1, 4 + u) for u in range(6)]
    + [(0, 10 + u) for u in range(5)] + [(1, 10 + u) for u in range(5)]
)


def _permute_block(xb, b0, b1):
    dsts = (b0, b1)

    def row(r, carry):
        vals = [xb[r, pl.ds(_L * u, _L)] for u in range(30)]
        for (oi, d), v in zip(_UNIT_MAP, vals):
            dsts[oi][r, pl.ds(_L * d, _L)] = v
        return carry

    lax.fori_loop(0, _R, row, 0)


def _sc_body(x, o0, o1, xb_a, xb_b, b0, b1, sem_in, sem_out):
    wid = lax.axis_index("s") * 2 + lax.axis_index("c")
    xbs = (xb_a, xb_b)

    def blk(step):
        return jnp.minimum(wid + _NW * step, _NBLK - 1)

    def in_copy(step, s):
        r0 = blk(step) * _R
        return pltpu.make_async_copy(x.at[pl.ds(r0, _R)], xbs[s],
                                     sem_in.at[s])

    def out_copies(step):
        r0 = blk(step) * _R
        return (
            pltpu.make_async_copy(b0, o0.at[pl.ds(r0, _R)], sem_out),
            pltpu.make_async_copy(b1, o1.at[pl.ds(r0, _R)], sem_out),
        )

    def half(step, s, refill):
        in_copy(step, s).wait()
        _permute_block(xbs[s], b0, b1)
        if refill:
            in_copy(step + 2, s).start()
        for c in out_copies(step):
            c.start()

    in_copy(0, 0).start()
    in_copy(1, 1).start()
    half(0, 0, True)

    def body(k, carry):
        s_b = 2 * k + 1
        for c in out_copies(s_b - 1):
            c.wait()
        half(s_b, 1, True)
        for c in out_copies(s_b):
            c.wait()
        half(s_b + 1, 0, True)
        return carry

    lax.fori_loop(0, _NSTEP // 2 - 2, body, 0)

    for c in out_copies(2 * (_NSTEP // 2 - 2)):
        c.wait()
    half(_NSTEP - 3, 1, True)
    for c in out_copies(_NSTEP - 3):
        c.wait()
    half(_NSTEP - 2, 0, False)
    for c in out_copies(_NSTEP - 2):
        c.wait()
    half(_NSTEP - 1, 1, False)
    for c in out_copies(_NSTEP - 1):
        c.wait()


def kernel(x):
    n, _ = x.shape
    run = pl.kernel(
        _sc_body,
        out_type=[jax.ShapeDtypeStruct((n, 240), jnp.float32)] * 2,
        mesh=plsc.VectorSubcoreMesh(core_axis_name="c", subcore_axis_name="s"),
        scratch_types=[
            pltpu.VMEM((_R, 480), jnp.float32),
            pltpu.VMEM((_R, 480), jnp.float32),
            pltpu.VMEM((_R, 240), jnp.float32),
            pltpu.VMEM((_R, 240), jnp.float32),
            pltpu.SemaphoreType.DMA((2,)),
            pltpu.SemaphoreType.DMA,
        ],
        compiler_params=pltpu.CompilerParams(use_tc_tiling_on_sc=True),
    )
    o0, o1 = run(x)
    return (o0, o1)


# speedup vs baseline: 1.0480x; 1.0480x over previous
import jax
import jax.numpy as jnp
from jax import lax
from jax.experimental import pallas as pl
from jax.experimental.pallas import tpu as pltpu
from jax.experimental.pallas import tpu_sc as plsc

_N = 100000
_NW = 32
_R = 80
_H = _R // 2
_NBLK = _N // _R
_NSTEP = -(-_NBLK // _NW)
_L = 16

_UNIT_MAP = (
    [(0, u) for u in range(4)] + [(1, u) for u in range(4)]
    + [(0, 4 + u) for u in range(6)] + [(1, 4 + u) for u in range(6)]
    + [(0, 10 + u) for u in range(5)] + [(1, 10 + u) for u in range(5)]
)


def _permute_half(xb, row0, b0, b1):
    dsts = (b0, b1)

    def row(r, carry):
        vals = [xb[row0 + r, pl.ds(_L * u, _L)] for u in range(30)]
        for (oi, d), v in zip(_UNIT_MAP, vals):
            dsts[oi][r, pl.ds(_L * d, _L)] = v
        return carry

    lax.fori_loop(0, _H, row, 0)


def _sc_body(x, o0, o1, xb_a, xb_b, b0a, b1a, b0b, b1b,
             sem_in, sem_out_a, sem_out_b):
    wid = lax.axis_index("s") * 2 + lax.axis_index("c")
    xbs = (xb_a, xb_b)

    def blk(step):
        return jnp.minimum(wid + _NW * step, _NBLK - 1)

    def in_copy(step, s):
        r0 = blk(step) * _R
        return pltpu.make_async_copy(x.at[pl.ds(r0, _R)], xbs[s],
                                     sem_in.at[s])

    def out_copies_a(step):
        r0 = blk(step) * _R
        return (
            pltpu.make_async_copy(b0a, o0.at[pl.ds(r0, _H)], sem_out_a),
            pltpu.make_async_copy(b1a, o1.at[pl.ds(r0, _H)], sem_out_a),
        )

    def out_copies_b(step):
        r0 = blk(step) * _R + _H
        return (
            pltpu.make_async_copy(b0b, o0.at[pl.ds(r0, _H)], sem_out_b),
            pltpu.make_async_copy(b1b, o1.at[pl.ds(r0, _H)], sem_out_b),
        )

    def half(step, s, first, refill):
        in_copy(step, s).wait()
        if not first:
            for c in out_copies_a(step):
                c.wait()
        _permute_half(xbs[s], 0, b0a, b1a)
        for c in out_copies_a(step):
            c.start()
        if not first:
            for c in out_copies_b(step):
                c.wait()
        _permute_half(xbs[s], _H, b0b, b1b)
        if refill:
            in_copy(step + 2, s).start()
        for c in out_copies_b(step):
            c.start()

    in_copy(0, 0).start()
    in_copy(1, 1).start()
    half(0, 0, True, True)

    def body(k, carry):
        half(2 * k + 1, 1, False, True)
        half(2 * k + 2, 0, False, True)
        return carry

    lax.fori_loop(0, _NSTEP // 2 - 2, body, 0)

    half(_NSTEP - 3, 1, False, True)
    half(_NSTEP - 2, 0, False, False)
    half(_NSTEP - 1, 1, False, False)
    for c in out_copies_a(_NSTEP - 1):
        c.wait()
    for c in out_copies_b(_NSTEP - 1):
        c.wait()


def kernel(x):
    n, _ = x.shape
    run = pl.kernel(
        _sc_body,
        out_type=[jax.ShapeDtypeStruct((n, 240), jnp.float32)] * 2,
        mesh=plsc.VectorSubcoreMesh(core_axis_name="c", subcore_axis_name="s"),
        scratch_types=[
            pltpu.VMEM((_R, 480), jnp.float32),
            pltpu.VMEM((_R, 480), jnp.float32),
            pltpu.VMEM((_H, 240), jnp.float32),
            pltpu.VMEM((_H, 240), jnp.float32),
            pltpu.VMEM((_H, 240), jnp.float32),
            pltpu.VMEM((_H, 240), jnp.float32),
            pltpu.SemaphoreType.DMA((2,)),
            pltpu.SemaphoreType.DMA,
            pltpu.SemaphoreType.DMA,
        ],
        compiler_params=pltpu.CompilerParams(use_tc_tiling_on_sc=True),
    )
    o0, o1 = run(x)
    return (o0, o1)
